# baseline (speedup 1.0000x reference)
import functools

import jax
import jax.numpy as jnp
from jax import lax
from jax.experimental import pallas as pl
from jax.experimental.pallas import tpu as pltpu
from jax.experimental.pallas import tpu_sc as plsc

NUM_EMB = 100000
DIM = 64
BATCH = 1024

NUM_CORES = 2
NUM_SUBCORES = 16
L = 16
NCHUNK = 8
CW = 12288
TAIL = NUM_EMB - NCHUNK * CW
TAIL_BASE = NCHUNK * CW
TAIL_W = 512
TAIL_W3 = TAIL - 3 * TAIL_W
SLABW = CW + TAIL_W
DPC = DIM // NUM_CORES
NGRP = DPC // 8
RPT = DPC // NUM_SUBCORES
NCOMP = 2960


def _sc_body(i1_hbm, i2_hbm, wt_hbm, out_hbm, scr_hbm,
             i1_v, i2_v, slab_v, part_v, ccol, cb, cscl,
             fb0, fb1, fb2, fb3, sem):
    s = lax.axis_index("s")
    c = lax.axis_index("c")
    glocal = s % NGRP
    owner = s // NGRP
    dbase = DPC * c + 8 * glocal

    pltpu.sync_copy(i1_hbm.at[pl.ds(0, BATCH)], i1_v)
    pltpu.sync_copy(i2_hbm.at[pl.ds(0, BATCH)], i2_v)

    iota = lax.iota(jnp.int32, L)
    zf = jnp.zeros((L,), jnp.float32)
    zi = jnp.zeros((L,), jnp.int32)

    def _zero(t, carry):
        for dl in range(8):
            part_v[dl, pl.ds(t * L, L)] = zf
        return carry

    def _compact(t, off, idx_ref, scaled, cbase, tbase, tw):
        idx = idx_ref[pl.ds(t * L, L)]
        local = idx - cbase
        m = (local >= 0) & (local < CW)
        if tbase is not None:
            lt = idx - tbase + CW
            mt = (lt >= CW) & (lt < CW + tw)
            m = m | mt
            local = jnp.where(mt, lt, local)
        if scaled:
            other = i1_v[pl.ds(t * L, L)]
            scl = jnp.where(idx != other, jnp.float32(1.0), jnp.float32(0.0))
        else:
            scl = jnp.full((L,), 1.0, jnp.float32)
        bvec = t * L + iota
        o = pl.multiple_of(off, 8)
        ccol[pl.ds(o, L)] = zi
        cb[pl.ds(o, L)] = zi
        cscl[pl.ds(o, L)] = zf
        plsc.store_compressed(ccol.at[pl.ds(o, L)], local, mask=m)
        plsc.store_compressed(cb.at[pl.ds(o, L)], bvec, mask=m)
        plsc.store_compressed(cscl.at[pl.ds(o, L)], scl, mask=m)
        n = plsc.all_reduce_population_count(m)[0]
        return off + ((n + 7) & ~7)

    def _gather(t, carry):
        cols = ccol[pl.ds(t * L, L)]
        bvec = cb[pl.ds(t * L, L)]
        scl = cscl[pl.ds(t * L, L)]
        for dl in range(8):
            dsplat = jnp.full((L,), dl, jnp.int32)
            v = plsc.load_gather(slab_v, [dsplat, cols])
            plsc.addupdate_scatter(part_v, [dsplat, bvec], v * scl)
        return carry

    for r in range(2):
        q = owner + NGRP * r
        cbase = pl.multiple_of(q * CW, 128)
        cp = pltpu.async_copy(
            wt_hbm.at[pl.ds(dbase, 8), pl.ds(cbase, CW)],
            slab_v.at[:, pl.ds(0, CW)], sem)

        if r == 1:
            tb = pl.multiple_of(TAIL_BASE + TAIL_W * owner, 128)
            tw = jnp.where(owner == NGRP - 1, TAIL_W3, TAIL_W)

            @pl.when(owner < NGRP - 1)
            def _():
                pltpu.sync_copy(
                    wt_hbm.at[pl.ds(dbase, 8), pl.ds(tb, TAIL_W)],
                    slab_v.at[:, pl.ds(CW, TAIL_W)])

            @pl.when(owner == NGRP - 1)
            def _():
                pltpu.sync_copy(
                    wt_hbm.at[pl.ds(dbase, 8), pl.ds(tb, 256)],
                    slab_v.at[:, pl.ds(CW, 256)])
        else:
            tb, tw = None, None
            lax.fori_loop(0, BATCH // L, _zero, 0)

        off = lax.fori_loop(
            0, BATCH // L,
            functools.partial(_compact, idx_ref=i1_v, scaled=False,
                              cbase=cbase, tbase=tb, tw=tw), 0)
        off = lax.fori_loop(
            0, BATCH // L,
            functools.partial(_compact, idx_ref=i2_v, scaled=True,
                              cbase=cbase, tbase=tb, tw=tw), off)
        o = pl.multiple_of(off, 8)
        ccol[pl.ds(o, L)] = zi
        cb[pl.ds(o, L)] = zi
        cscl[pl.ds(o, L)] = zf
        cp.wait()
        lax.fori_loop(0, (off + L - 1) // L, _gather, 0)

    pltpu.sync_copy(part_v, scr_hbm.at[c, owner, pl.ds(8 * glocal, 8), :])
    plsc.subcore_barrier()

    rbase = RPT * s
    pltpu.sync_copy(scr_hbm.at[c, 0, pl.ds(rbase, RPT), :], fb0)
    pltpu.sync_copy(scr_hbm.at[c, 1, pl.ds(rbase, RPT), :], fb1)
    pltpu.sync_copy(scr_hbm.at[c, 2, pl.ds(rbase, RPT), :], fb2)
    pltpu.sync_copy(scr_hbm.at[c, 3, pl.ds(rbase, RPT), :], fb3)

    def _reduce(t, carry):
        for row in range(RPT):
            sl = pl.ds(t * L, L)
            fb0[row, sl] = ((fb0[row, sl] + fb1[row, sl])
                            + (fb2[row, sl] + fb3[row, sl]))
        return carry
    lax.fori_loop(0, BATCH // L, _reduce, 0)

    pltpu.sync_copy(fb0, out_hbm.at[pl.ds(DPC * c + rbase, RPT), :])


_two_hot_sc = functools.partial(
    pl.kernel,
    out_type=(jax.ShapeDtypeStruct((DIM, BATCH), jnp.float32),
              jax.ShapeDtypeStruct((NUM_CORES, NGRP, DPC, BATCH),
                                   jnp.float32)),
    mesh=plsc.VectorSubcoreMesh(core_axis_name="c", subcore_axis_name="s"),
    compiler_params=pltpu.CompilerParams(needs_layout_passes=False),
    scratch_types=[
        pltpu.VMEM((BATCH,), jnp.int32),
        pltpu.VMEM((BATCH,), jnp.int32),
        pltpu.VMEM((8, SLABW), jnp.float32),
        pltpu.VMEM((8, BATCH), jnp.float32),
        pltpu.VMEM((NCOMP,), jnp.int32),
        pltpu.VMEM((NCOMP,), jnp.int32),
        pltpu.VMEM((NCOMP,), jnp.float32),
        pltpu.VMEM((RPT, BATCH), jnp.float32),
        pltpu.VMEM((RPT, BATCH), jnp.float32),
        pltpu.VMEM((RPT, BATCH), jnp.float32),
        pltpu.VMEM((RPT, BATCH), jnp.float32),
        pltpu.SemaphoreType.DMA,
    ],
)(_sc_body)


@jax.jit
def kernel(input_one, input_two, weight):
    i1 = input_one.astype(jnp.int32)
    i2 = input_two.astype(jnp.int32)
    out_t, _ = _two_hot_sc(i1, i2, weight.T)
    return out_t.T


# speedup vs baseline: 1.1777x; 1.1777x over previous
import functools

import jax
import jax.numpy as jnp
from jax import lax
from jax.experimental import pallas as pl
from jax.experimental.pallas import tpu as pltpu
from jax.experimental.pallas import tpu_sc as plsc

NUM_EMB = 100000
DIM = 64
BATCH = 1024

NUM_CORES = 2
NUM_SUBCORES = 16
L = 16
NCHUNK = 8
CW = 12544
CW_LAST = NUM_EMB - (NCHUNK - 1) * CW
CW_LAST_DMA = ((CW_LAST + 127) // 128) * 128
DPC = DIM // NUM_CORES
NGRP = DPC // 8
ROWS_PER_TILE = DPC // NUM_SUBCORES


def _sc_body(i1_hbm, i2_hbm, wt_hbm, out_hbm,
             i1_v, i2_v, scale_v, slab_v, part_v,
             fb0, fb1, fb2, fb3, out_v, shared, sem):
    s = lax.axis_index("s")
    c = lax.axis_index("c")
    glocal = s % NGRP
    owner = s // NGRP
    dbase = DPC * c + 8 * glocal

    pltpu.sync_copy(i1_hbm.at[pl.ds(0, BATCH)], i1_v)
    pltpu.sync_copy(i2_hbm.at[pl.ds(0, BATCH)], i2_v)

    iota = lax.iota(jnp.int32, L)

    def _prep(t, carry):
        a = i1_v[pl.ds(t * L, L)]
        b = i2_v[pl.ds(t * L, L)]
        scale_v[pl.ds(t * L, L)] = jnp.where(
            a != b, jnp.float32(1.0), jnp.float32(0.0))
        z = jnp.zeros((L,), jnp.float32)
        for dl in range(8):
            part_v[dl, pl.ds(t * L, L)] = z
        return carry

    for r in range(2):
        q = owner + NGRP * r
        cbase = pl.multiple_of(q * CW, 128)
        w = jnp.where(q == NCHUNK - 1, CW_LAST, CW).astype(jnp.int32)

        if r == 0:
            cp = pltpu.async_copy(
                wt_hbm.at[pl.ds(dbase, 8), pl.ds(cbase, CW)], slab_v, sem)
            lax.fori_loop(0, BATCH // L, _prep, 0)
            cp.wait()
        else:
            @pl.when(q == NCHUNK - 1)
            def _():
                pltpu.sync_copy(
                    wt_hbm.at[pl.ds(dbase, 8), pl.ds(cbase, CW_LAST_DMA)],
                    slab_v.at[:, pl.ds(0, CW_LAST_DMA)])

            @pl.when(q != NCHUNK - 1)
            def _():
                pltpu.sync_copy(
                    wt_hbm.at[pl.ds(dbase, 8), pl.ds(cbase, CW)], slab_v)

        def _accum(t, carry, idx_ref, scaled):
            idx = idx_ref[pl.ds(t * L, L)]
            local = idx - cbase
            m = (local >= 0) & (local < w)
            bvec = t * L + iota
            if scaled:
                scl = scale_v[pl.ds(t * L, L)]
            for dl in range(8):
                dsplat = jnp.full((L,), dl, jnp.int32)
                v = plsc.load_gather(slab_v, [dsplat, local], mask=m)
                if scaled:
                    v = v * scl
                plsc.addupdate_scatter(part_v, [dsplat, bvec], v, mask=m)
            return carry

        lax.fori_loop(0, BATCH // L,
                      functools.partial(_accum, idx_ref=i1_v, scaled=False), 0)
        lax.fori_loop(0, BATCH // L,
                      functools.partial(_accum, idx_ref=i2_v, scaled=True), 0)

    pltpu.sync_copy(part_v, shared.at[owner, pl.ds(8 * glocal, 8), :])
    plsc.subcore_barrier()

    rbase = ROWS_PER_TILE * s
    pltpu.sync_copy(shared.at[0, pl.ds(rbase, ROWS_PER_TILE), :], fb0)
    pltpu.sync_copy(shared.at[1, pl.ds(rbase, ROWS_PER_TILE), :], fb1)
    pltpu.sync_copy(shared.at[2, pl.ds(rbase, ROWS_PER_TILE), :], fb2)
    pltpu.sync_copy(shared.at[3, pl.ds(rbase, ROWS_PER_TILE), :], fb3)

    def _reduce(t, carry):
        for row in range(ROWS_PER_TILE):
            sl = pl.ds(t * L, L)
            out_v[row, sl] = ((fb0[row, sl] + fb1[row, sl])
                              + (fb2[row, sl] + fb3[row, sl]))
        return carry
    lax.fori_loop(0, BATCH // L, _reduce, 0)

    pltpu.sync_copy(out_v, out_hbm.at[pl.ds(DPC * c + rbase, ROWS_PER_TILE), :])


_two_hot_sc = functools.partial(
    pl.kernel,
    out_type=jax.ShapeDtypeStruct((DIM, BATCH), jnp.float32),
    mesh=plsc.VectorSubcoreMesh(core_axis_name="c", subcore_axis_name="s"),
    compiler_params=pltpu.CompilerParams(needs_layout_passes=False),
    scratch_types=[
        pltpu.VMEM((BATCH,), jnp.int32),
        pltpu.VMEM((BATCH,), jnp.int32),
        pltpu.VMEM((BATCH,), jnp.float32),
        pltpu.VMEM((8, CW), jnp.float32),
        pltpu.VMEM((8, BATCH), jnp.float32),
        pltpu.VMEM((ROWS_PER_TILE, BATCH), jnp.float32),
        pltpu.VMEM((ROWS_PER_TILE, BATCH), jnp.float32),
        pltpu.VMEM((ROWS_PER_TILE, BATCH), jnp.float32),
        pltpu.VMEM((ROWS_PER_TILE, BATCH), jnp.float32),
        pltpu.VMEM((ROWS_PER_TILE, BATCH), jnp.float32),
        pltpu.VMEM_SHARED((NGRP, DPC, BATCH), jnp.float32),
        pltpu.SemaphoreType.DMA,
    ],
)(_sc_body)


@jax.jit
def kernel(input_one, input_two, weight):
    i1 = input_one.astype(jnp.int32)
    i2 = input_two.astype(jnp.int32)
    return _two_hot_sc(i1, i2, weight.T).T
